# baseline (speedup 1.0000x reference)
import functools

import jax
import jax.numpy as jnp
from jax import lax
from jax.experimental import pallas as pl
from jax.experimental.pallas import tpu as pltpu
from jax.experimental.pallas import tpu_sc as plsc

_LANES = 16
_HI = -65536


def _round_bf16_hi(x):
    b = lax.bitcast_convert_type(x, jnp.int32)
    return b + 0x7FFF + (lax.shift_right_logical(b, 16) & 1)


@functools.lru_cache(maxsize=None)
def _make_tc_packer(vocab, emb, cb):
    assert emb == 64
    nblk = (vocab + cb - 1) // cb
    q = cb // 4

    def body(t_ref, out_ref):
        x = t_ref[...]
        hi = _round_bf16_hi(x[0:32, :]) & _HI
        lo = lax.shift_right_logical(_round_bf16_hi(x[32:64, :]), 16)
        w = hi | lo
        wp4 = jnp.concatenate(
            [w[:, 0:q], w[:, q:2 * q], w[:, 2 * q:3 * q], w[:, 3 * q:4 * q]],
            axis=0)
        out_ref[...] = wp4.T

    return pl.pallas_call(
        body,
        grid=(nblk,),
        in_specs=[pl.BlockSpec((emb, cb), lambda i: (0, i))],
        out_specs=pl.BlockSpec((q, 128), lambda i: (i, 0)),
        out_shape=jax.ShapeDtypeStruct((nblk * q, 128), jnp.int32),
    )


@functools.lru_cache(maxsize=None)
def _make_sc_pool(vocab_pad, emb, batch, seqlen, nc, ns):
    nw = nc * ns
    assert batch % nw == 0 and emb == 64
    s_per = batch // nw
    tokens = s_per * seqlen
    assert tokens % 128 == 0
    nchunk = tokens // 128
    words = emb // 2
    import math
    pat_tok = seqlen * 128 // math.gcd(seqlen, 128)
    pat = pat_tok // 128
    pat_sent = pat_tok // seqlen
    reps = tokens // pat_tok
    nring = 10
    per_iter = 2
    assert (per_iter * pat) % nring == 0 and reps % per_iter == 0
    mesh = plsc.VectorSubcoreMesh(core_axis_name="c", subcore_axis_name="s")

    @functools.partial(
        pl.kernel,
        out_type=jax.ShapeDtypeStruct((batch, emb), jnp.float32),
        mesh=mesh,
        scratch_types=[
            pltpu.VMEM((nchunk, 128), jnp.int32),
            pltpu.VMEM((nring, 128, words), jnp.int32),
            pltpu.VMEM((s_per, emb), jnp.float32),
        ] + [pltpu.SemaphoreType.DMA] * nring,
        compiler_params=pltpu.CompilerParams(use_tc_tiling_on_sc=False,
                                             needs_layout_passes=False),
    )
    def sc_pool(idx_hbm, table_hbm, sums_hbm, idx_v, rows_v, sums_v, *sems):
        wid = lax.axis_index("s") * nc + lax.axis_index("c")
        pltpu.sync_copy(idx_hbm.at[pl.ds(wid * nchunk, nchunk)], idx_v)
        slots = [rows_v.at[r] for r in range(nring)]

        def fire(j, dst, sem):
            pltpu.async_copy(table_hbm.at[idx_v.at[j]], dst, sem)

        def wait(j, dst, sem):
            pltpu.make_async_copy(table_hbm.at[idx_v.at[j]], dst, sem).wait()

        def accum_range(rows, a, b, acc):
            def body(i, carry):
                carry = list(carry)
                r = a + i * 8
                for k in range(8):
                    off = (k % 2) * 4
                    w0 = rows[r + k, pl.ds(0, _LANES)]
                    w1 = rows[r + k, pl.ds(_LANES, _LANES)]
                    carry[off + 0] += plsc.bitcast(w0, jnp.float32)
                    carry[off + 1] += plsc.bitcast(w1, jnp.float32)
                    carry[off + 2] += plsc.bitcast(
                        lax.shift_left(w0, 16), jnp.float32)
                    carry[off + 3] += plsc.bitcast(
                        lax.shift_left(w1, 16), jnp.float32)
                return tuple(carry)

            return lax.fori_loop(0, (b - a) // 8, body, acc)

        zero8 = (jnp.zeros((_LANES,), jnp.float32),) * 8
        for r in range(nring):
            fire(r, slots[r], sems[r])

        def store(s, acc):
            sums_v[s, pl.ds(0, _LANES)] = acc[0] + acc[4]
            sums_v[s, pl.ds(_LANES, _LANES)] = acc[1] + acc[5]
            sums_v[s, pl.ds(2 * _LANES, _LANES)] = acc[2] + acc[6]
            sums_v[s, pl.ds(3 * _LANES, _LANES)] = acc[3] + acc[7]

        npat = per_iter * pat

        def rep(i, carry):
            jb = npat * i
            sb = per_iter * pat_sent * i
            acc = zero8
            for rr in range(npat):
                slot = rr % nring
                wait(jb + rr, slots[slot], sems[slot])
                r = rr % pat
                sb2 = sb + pat_sent * (rr // pat)
                start = 128 * r
                cut = seqlen - start % seqlen
                if cut <= 128:
                    acc = accum_range(slots[slot], 0, cut, acc)
                    store(sb2 + start // seqlen, acc)
                    acc = zero8
                    if cut < 128:
                        acc = accum_range(slots[slot], cut, 128, acc)
                else:
                    acc = accum_range(slots[slot], 0, 128, acc)
                if rr < npat - nring:
                    fire(jb + rr + nring, slots[slot], sems[slot])
                else:
                    @pl.when(i < reps // per_iter - 1)
                    def _():
                        fire(jb + rr + nring, slots[slot], sems[slot])
            return carry

        lax.fori_loop(0, reps // per_iter, rep, 0)
        pltpu.sync_copy(sums_v, sums_hbm.at[pl.ds(wid * s_per, s_per)])

    return sc_pool


@functools.lru_cache(maxsize=None)
def _make_tc_head(batch, seqlen, emb, nclass, pad):
    bb = 512
    assert batch % bb == 0

    def body(sent_ref, sums_ref, w_ref, b_ref, out_ref):
        cnt = jnp.sum((sent_ref[...] != pad).astype(jnp.float32), axis=1,
                      keepdims=True)
        pooled = sums_ref[...] / cnt
        logits = lax.dot_general(pooled, w_ref[...], (((1,), (1,)), ((), ())),
                                 preferred_element_type=jnp.float32)
        logits = logits + b_ref[...]
        shifted = logits - jnp.max(logits, axis=1, keepdims=True)
        lse = jnp.log(jnp.sum(jnp.exp(shifted), axis=1, keepdims=True))
        out_ref[...] = shifted - lse

    return pl.pallas_call(
        body,
        grid=(batch // bb,),
        in_specs=[
            pl.BlockSpec((bb, seqlen), lambda i: (i, 0)),
            pl.BlockSpec((bb, emb), lambda i: (i, 0)),
            pl.BlockSpec((nclass, emb), lambda i: (0, 0)),
            pl.BlockSpec((1, nclass), lambda i: (0, 0)),
        ],
        out_specs=pl.BlockSpec((bb, nclass), lambda i: (i, 0)),
        out_shape=jax.ShapeDtypeStruct((batch, nclass), jnp.float32),
    )


def kernel(sentences, emb_table, fc_w, fc_b):
    batch, seqlen = sentences.shape
    vocab, emb = emb_table.shape
    nclass = fc_w.shape[0]
    info = plsc.get_sparse_core_info()
    nc, ns = info.num_cores, info.num_subcores
    sent_i32 = sentences.astype(jnp.int32)
    cb = 65536
    q = cb // 4
    nblk = (vocab + cb - 1) // cb
    vocab_pad = nblk * cb
    packed = _make_tc_packer(vocab, emb, cb)(emb_table.T)
    table_lin = packed.reshape(vocab_pad, emb // 2)
    lo = sent_i32 % cb
    idx_t = (sent_i32 - lo) + 4 * (lo % q) + lo // q
    idx_flat = lax.optimization_barrier(
        idx_t.reshape(batch * seqlen // 128, 128))
    sums = _make_sc_pool(vocab_pad, emb, batch, seqlen, nc, ns)(
        idx_flat, table_lin)
    head = _make_tc_head(batch, seqlen, emb, nclass, 0)
    return head(sent_i32, sums, fc_w, fc_b.reshape(1, nclass))


# speedup vs baseline: 1.0056x; 1.0056x over previous
import functools

import jax
import jax.numpy as jnp
from jax import lax
from jax.experimental import pallas as pl
from jax.experimental.pallas import tpu as pltpu
from jax.experimental.pallas import tpu_sc as plsc

_LANES = 16
_HI = -65536


def _round_bf16_hi(x):
    b = lax.bitcast_convert_type(x, jnp.int32)
    return b + 0x7FFF + (lax.shift_right_logical(b, 16) & 1)


@functools.lru_cache(maxsize=None)
def _make_tc_packer(vocab, emb, cb):
    assert emb == 64
    nblk = (vocab + cb - 1) // cb
    q = cb // 4

    def body(t_ref, out_ref):
        x = t_ref[...]
        hi = _round_bf16_hi(x[0:32, :]) & _HI
        lo = lax.shift_right_logical(_round_bf16_hi(x[32:64, :]), 16)
        w = hi | lo
        wp4 = jnp.concatenate(
            [w[:, 0:q], w[:, q:2 * q], w[:, 2 * q:3 * q], w[:, 3 * q:4 * q]],
            axis=0)
        out_ref[...] = wp4.T

    return pl.pallas_call(
        body,
        grid=(nblk,),
        in_specs=[pl.BlockSpec((emb, cb), lambda i: (0, i))],
        out_specs=pl.BlockSpec((q, 128), lambda i: (i, 0)),
        out_shape=jax.ShapeDtypeStruct((nblk * q, 128), jnp.int32),
    )


@functools.lru_cache(maxsize=None)
def _make_sc_pool(vocab_pad, emb, batch, seqlen, nc, ns):
    nw = nc * ns
    assert batch % nw == 0 and emb == 64
    s_per = batch // nw
    tokens = s_per * seqlen
    assert tokens % 128 == 0
    nchunk = tokens // 128
    words = emb // 2
    import math
    pat_tok = seqlen * 128 // math.gcd(seqlen, 128)
    pat = pat_tok // 128
    pat_sent = pat_tok // seqlen
    reps = tokens // pat_tok
    nring = 10
    per_iter = 2
    assert (per_iter * pat) % nring == 0 and reps % per_iter == 0
    mesh = plsc.VectorSubcoreMesh(core_axis_name="c", subcore_axis_name="s")

    @functools.partial(
        pl.kernel,
        out_type=jax.ShapeDtypeStruct((batch, emb), jnp.float32),
        mesh=mesh,
        scratch_types=[
            pltpu.VMEM((nchunk, 128), jnp.int32),
            pltpu.VMEM((nring, 128, words), jnp.int32),
            pltpu.VMEM((s_per, emb), jnp.float32),
        ] + [pltpu.SemaphoreType.DMA] * nring,
        compiler_params=pltpu.CompilerParams(use_tc_tiling_on_sc=False,
                                             needs_layout_passes=False),
    )
    def sc_pool(idx_hbm, table_hbm, sums_hbm, idx_v, rows_v, sums_v, *sems):
        wid = lax.axis_index("s") * nc + lax.axis_index("c")
        pltpu.sync_copy(idx_hbm.at[pl.ds(wid * nchunk, nchunk)], idx_v)
        slots = [rows_v.at[r] for r in range(nring)]

        def fire(j, dst, sem):
            pltpu.async_copy(table_hbm.at[idx_v.at[j]], dst, sem)

        def wait(j, dst, sem):
            pltpu.make_async_copy(table_hbm.at[idx_v.at[j]], dst, sem).wait()

        def accum_range(rows, a, b, acc):
            def body(i, carry):
                carry = list(carry)
                r = a + i * 8
                for k in range(8):
                    off = (k % 2) * 4
                    w0 = rows[r + k, pl.ds(0, _LANES)]
                    w1 = rows[r + k, pl.ds(_LANES, _LANES)]
                    carry[off + 0] += plsc.bitcast(w0, jnp.float32)
                    carry[off + 1] += plsc.bitcast(w1, jnp.float32)
                    carry[off + 2] += plsc.bitcast(
                        lax.shift_left(w0, 16), jnp.float32)
                    carry[off + 3] += plsc.bitcast(
                        lax.shift_left(w1, 16), jnp.float32)
                return tuple(carry)

            return lax.fori_loop(0, (b - a) // 8, body, acc)

        zero8 = (jnp.zeros((_LANES,), jnp.float32),) * 8
        for r in range(nring):
            fire(r, slots[r], sems[r])

        def store(s, acc):
            sums_v[s, pl.ds(0, _LANES)] = acc[0] + acc[4]
            sums_v[s, pl.ds(_LANES, _LANES)] = acc[1] + acc[5]
            sums_v[s, pl.ds(2 * _LANES, _LANES)] = acc[2] + acc[6]
            sums_v[s, pl.ds(3 * _LANES, _LANES)] = acc[3] + acc[7]

        npat = per_iter * pat

        def rep(i, carry):
            jb = npat * i
            sb = per_iter * pat_sent * i
            acc = zero8
            for rr in range(npat):
                slot = rr % nring
                wait(jb + rr, slots[slot], sems[slot])
                r = rr % pat
                sb2 = sb + pat_sent * (rr // pat)
                start = 128 * r
                cut = seqlen - start % seqlen
                if cut <= 128:
                    acc = accum_range(slots[slot], 0, cut, acc)
                    store(sb2 + start // seqlen, acc)
                    acc = zero8
                    if cut < 128:
                        acc = accum_range(slots[slot], cut, 128, acc)
                else:
                    acc = accum_range(slots[slot], 0, 128, acc)
                if rr < npat - nring:
                    fire(jb + rr + nring, slots[slot], sems[slot])
                else:
                    @pl.when(i < reps // per_iter - 1)
                    def _():
                        fire(jb + rr + nring, slots[slot], sems[slot])
            return carry

        lax.fori_loop(0, reps // per_iter, rep, 0)
        pltpu.sync_copy(sums_v, sums_hbm.at[pl.ds(wid * s_per, s_per)])

    return sc_pool


@functools.lru_cache(maxsize=None)
def _make_tc_head(batch, seqlen, emb, nclass, pad):
    bb = 512
    assert batch % bb == 0

    def body(sent_ref, sums_ref, w_ref, b_ref, out_ref):
        cnt = jnp.sum((sent_ref[...] != pad).astype(jnp.float32), axis=1,
                      keepdims=True)
        pooled = sums_ref[...] / cnt
        logits = lax.dot_general(pooled, w_ref[...], (((1,), (1,)), ((), ())),
                                 preferred_element_type=jnp.float32)
        logits = logits + b_ref[...]
        shifted = logits - jnp.max(logits, axis=1, keepdims=True)
        lse = jnp.log(jnp.sum(jnp.exp(shifted), axis=1, keepdims=True))
        out_ref[...] = shifted - lse

    return pl.pallas_call(
        body,
        grid=(batch // bb,),
        in_specs=[
            pl.BlockSpec((bb, seqlen), lambda i: (i, 0)),
            pl.BlockSpec((bb, emb), lambda i: (i, 0)),
            pl.BlockSpec((nclass, emb), lambda i: (0, 0)),
            pl.BlockSpec((1, nclass), lambda i: (0, 0)),
        ],
        out_specs=pl.BlockSpec((bb, nclass), lambda i: (i, 0)),
        out_shape=jax.ShapeDtypeStruct((batch, nclass), jnp.float32),
    )


def kernel(sentences, emb_table, fc_w, fc_b):
    batch, seqlen = sentences.shape
    vocab, emb = emb_table.shape
    nclass = fc_w.shape[0]
    info = plsc.get_sparse_core_info()
    nc, ns = info.num_cores, info.num_subcores
    sent_i32 = sentences.astype(jnp.int32)
    cb = 32768
    q = cb // 4
    nblk = (vocab + cb - 1) // cb
    vocab_pad = nblk * cb
    packed = _make_tc_packer(vocab, emb, cb)(emb_table.T)
    table_lin = packed.reshape(vocab_pad, emb // 2)
    lo = sent_i32 % cb
    idx_t = (sent_i32 - lo) + 4 * (lo % q) + lo // q
    idx_flat = lax.optimization_barrier(
        idx_t.reshape(batch * seqlen // 128, 128))
    sums = _make_sc_pool(vocab_pad, emb, batch, seqlen, nc, ns)(
        idx_flat, table_lin)
    head = _make_tc_head(batch, seqlen, emb, nclass, 0)
    return head(sent_i32, sums, fc_w, fc_b.reshape(1, nclass))
